# baseline (speedup 1.0000x reference)
import functools

import jax
import jax.numpy as jnp
from jax import lax
from jax.experimental import pallas as pl
from jax.experimental.pallas import tpu as pltpu
from jax.experimental.pallas import tpu_sc as plsc

B, P, N, S = 16, 5000, 100, 8
L = 16
PH = 2512
H1OFF = P - PH
NPAD = 112
NCHUNK = PH // L
TCH = NPAD // L
TGB = 4
NW = 32


def _sc_body(px0h, py0h, px1h, py1h, tx0h, ty0h, tx1h, ty1h,
             scoresh, ctxh, scalesf,
             pbest, pidx, pbox, pscale, pctx, pscore,
             px0, py0, px1, py1, a1,
             tx0, ty0, tx1, ty1, a2,
             bval, bidx, gidx, gidx2, buf14, sem):
    c = lax.axis_index("c")
    s = lax.axis_index("s")
    b = c * 8 + s // 2
    h = s % 2
    w = c * 16 + s
    poff = b * P + h * H1OFF
    toff = b * NPAD

    pltpu.sync_copy(px0h.at[pl.ds(poff, PH)], px0)
    pltpu.sync_copy(py0h.at[pl.ds(poff, PH)], py0)
    pltpu.sync_copy(px1h.at[pl.ds(poff, PH)], px1)
    pltpu.sync_copy(py1h.at[pl.ds(poff, PH)], py1)
    pltpu.sync_copy(tx0h.at[pl.ds(toff, NPAD)], tx0)
    pltpu.sync_copy(ty0h.at[pl.ds(toff, NPAD)], ty0)
    pltpu.sync_copy(tx1h.at[pl.ds(toff, NPAD)], tx1)
    pltpu.sync_copy(ty1h.at[pl.ds(toff, NPAD)], ty1)

    lane = lax.iota(jnp.int32, L)

    def _area1(i, _):
        sl = pl.ds(i * L, L)
        a1[sl] = (px1[sl] - px0[sl]) * (py1[sl] - py0[sl])
        return 0
    lax.fori_loop(0, NCHUNK, _area1, 0)

    def _area2(i, _):
        sl = pl.ds(i * L, L)
        a2[sl] = (tx1[sl] - tx0[sl]) * (ty1[sl] - ty0[sl])
        return 0
    lax.fori_loop(0, TCH, _area2, 0)

    minf = jnp.full((L,), -jnp.inf, jnp.float32)
    zi = jnp.zeros((L,), jnp.int32)

    def _per_tchunk(tc, _):
        tsl = pl.ds(tc * L, L)
        v0 = tx0[tsl]
        v1 = ty0[tsl]
        v2 = tx1[tsl]
        v3 = ty1[tsl]
        va = a2[tsl]

        def _per_group(g, carry):
            resv, residx = carry
            t0s, t1s, t2s, t3s, a2s = [], [], [], [], []
            for jj in range(TGB):
                sel = lane == (g * TGB + jj)
                t0s.append(jnp.sum(jnp.where(sel, v0, 0.0)))
                t1s.append(jnp.sum(jnp.where(sel, v1, 0.0)))
                t2s.append(jnp.sum(jnp.where(sel, v2, 0.0)))
                t3s.append(jnp.sum(jnp.where(sel, v3, 0.0)))
                a2s.append(jnp.sum(jnp.where(sel, va, 0.0)))

            def _chunk(k, carry2):
                ms = list(carry2[:TGB])
                bis = list(carry2[TGB:])
                sl = pl.ds(k * L, L)
                x0 = px0[sl]
                y0 = py0[sl]
                x1 = px1[sl]
                y1 = py1[sl]
                ar = a1[sl]
                for jj in range(TGB):
                    wx = jnp.maximum(
                        jnp.minimum(x1, t2s[jj]) - jnp.maximum(x0, t0s[jj]),
                        0.0)
                    wy = jnp.maximum(
                        jnp.minimum(y1, t3s[jj]) - jnp.maximum(y0, t1s[jj]),
                        0.0)
                    inter = wx * wy
                    iou = inter / ((ar + a2s[jj]) - inter)
                    gt = iou > ms[jj]
                    ms[jj] = jnp.where(gt, iou, ms[jj])
                    bis[jj] = jnp.where(gt, k, bis[jj])
                return tuple(ms) + tuple(bis)

            init = (minf,) * TGB + (zi,) * TGB
            out = lax.fori_loop(0, NCHUNK, _chunk, init)
            for jj in range(TGB):
                m = out[jj]
                bi = out[TGB + jj]
                vmax = jnp.max(m)
                pc = jnp.where(m == vmax, bi * L + lane, jnp.int32(2 ** 30))
                pmin = jnp.min(pc)
                pmin = jnp.where(pmin >= 2 ** 30, jnp.int32(0), pmin)
                sel = lane == (g * TGB + jj)
                resv = jnp.where(sel, vmax, resv)
                residx = jnp.where(sel, pmin + h * H1OFF, residx)
            return resv, residx

        z = jnp.zeros((L,), jnp.float32)
        resv, residx = lax.fori_loop(0, L // TGB, _per_group, (z, zi))
        bval[tsl] = resv
        bidx[tsl] = residx
        gidx[tsl] = residx + b * P
        return 0

    lax.fori_loop(0, TCH, _per_tchunk, 0)

    pltpu.sync_copy(bval, pbest.at[w])
    pltpu.sync_copy(bidx, pidx.at[w])

    for jj in range(8):
        def _w(i, _, jj=jj):
            sl = pl.ds(i * L, L)
            gidx2[jj, sl] = gidx[sl] * 8 + jj
            return 0
        lax.fori_loop(0, TCH, _w, 0)

    copies = []

    def _fire(table, idx_ref, j):
        copies.append(
            pltpu.make_async_copy(table.at[idx_ref], buf14.at[j], sem))
        copies[-1].start()

    _fire(scoresh, gidx, 0)
    _fire(ctxh, gidx, 1)
    for cc, tbl in enumerate((px0h, py0h, px1h, py1h)):
        _fire(tbl, gidx, 2 + cc)
    for jj in range(8):
        _fire(scalesf, gidx2.at[jj], 6 + jj)
    for cp in copies:
        cp.wait()

    pltpu.sync_copy(buf14.at[0], pscore.at[w])
    pltpu.sync_copy(buf14.at[1], pctx.at[w])
    for cc in range(4):
        pltpu.sync_copy(buf14.at[2 + cc], pbox.at[w, cc])
    for jj in range(8):
        pltpu.sync_copy(buf14.at[6 + jj], pscale.at[w, jj])


_sc_kernel = functools.partial(
    pl.kernel,
    out_type=[
        jax.ShapeDtypeStruct((NW, NPAD), jnp.float32),
        jax.ShapeDtypeStruct((NW, NPAD), jnp.int32),
        jax.ShapeDtypeStruct((NW, 4, NPAD), jnp.float32),
        jax.ShapeDtypeStruct((NW, 8, NPAD), jnp.float32),
        jax.ShapeDtypeStruct((NW, NPAD), jnp.float32),
        jax.ShapeDtypeStruct((NW, NPAD), jnp.float32),
    ],
    mesh=plsc.VectorSubcoreMesh(core_axis_name="c", subcore_axis_name="s",
                                num_cores=2, num_subcores=16),
    compiler_params=pltpu.CompilerParams(needs_layout_passes=False),
    scratch_types=[
        pltpu.VMEM((PH,), jnp.float32),
        pltpu.VMEM((PH,), jnp.float32),
        pltpu.VMEM((PH,), jnp.float32),
        pltpu.VMEM((PH,), jnp.float32),
        pltpu.VMEM((PH,), jnp.float32),
        pltpu.VMEM((NPAD,), jnp.float32),
        pltpu.VMEM((NPAD,), jnp.float32),
        pltpu.VMEM((NPAD,), jnp.float32),
        pltpu.VMEM((NPAD,), jnp.float32),
        pltpu.VMEM((NPAD,), jnp.float32),
        pltpu.VMEM((NPAD,), jnp.float32),
        pltpu.VMEM((NPAD,), jnp.int32),
        pltpu.VMEM((NPAD,), jnp.int32),
        pltpu.VMEM((8, NPAD), jnp.int32),
        pltpu.VMEM((14, NPAD), jnp.float32),
        pltpu.SemaphoreType.DMA,
    ],
)(_sc_body)


def _tc_body(scores_ref, ctx_ref, scalesf_ref, boxesf_ref,
             pbest_ref, pidx_ref, pbestT_ref, pidxT_ref, pscoreT_ref,
             pbox_ref, pscale_ref, pctx_ref,
             tbox_ref, tscf_ref, tctx_ref,
             res_ref, acc_ref):
    i = pl.program_id(0)

    @pl.when(i == 0)
    def _init():
        res_ref[...] = jnp.zeros((8, 128), jnp.float32)
        acc_ref[0] = 0.0
        acc_ref[1] = 0.0
        acc_ref[2] = 0.0
        acc_ref[3] = 0.0

    sc = scores_ref[0, 0, :]
    bce0 = jnp.maximum(sc, 0.0) + jnp.log1p(jnp.exp(-jnp.abs(sc)))
    sum_bce0 = jnp.sum(bce0)

    v0 = pbest_ref[0, 0, :]
    v1 = pbest_ref[0, 1, :]
    gt = v1 > v0
    best = jnp.where(gt, v1, v0)
    nmask = lax.broadcasted_iota(jnp.int32, (NPAD,), 0) < N
    valid = (best > 0.5) & nmask
    vf = valid.astype(jnp.float32)
    cnt = jnp.sum(vf)
    cnt_s = jnp.maximum(cnt, 1.0)

    @pl.when(cnt > 0.0)
    def _valid_branch():
        gtb = gt[None, :]
        sel_box = jnp.where(gtb, pbox_ref[0, 1], pbox_ref[0, 0])
        tbox = tbox_ref[0]
        d = jnp.abs(sel_box - tbox)
        sl1 = jnp.where(d < 0.1, 5.0 * d * d, d - 0.05)
        bl = jnp.sum(sl1, axis=0) * best
        box_v = jnp.sum(bl * vf) / (cnt_s * 4.0)

        osc = jnp.where(gtb, pscale_ref[0, 1], pscale_ref[0, 0])
        mx = jnp.max(osc, axis=0, keepdims=True)
        lse = mx[0] + jnp.log(jnp.sum(jnp.exp(osc - mx), axis=0))
        srange = lax.broadcasted_iota(jnp.int32, (S, NPAD), 0).astype(jnp.float32)
        oh = (srange == tscf_ref[0]).astype(jnp.float32)
        picked = jnp.sum(osc * oh, axis=0)
        scale_v = jnp.sum((lse - picked) * vf) / cnt_s

        octx = jnp.where(gt, pctx_ref[0, 1], pctx_ref[0, 0])
        tctx = tctx_ref[0, 0, :]
        cbce = (jnp.maximum(octx, 0.0) - octx * tctx
                + jnp.log1p(jnp.exp(-jnp.abs(octx))))
        ctx_v = jnp.sum(cbce * vf) / cnt_s

        bT0 = pbestT_ref[0, :, 0:1]
        bT1 = pbestT_ref[0, :, 1:2]
        gtT = bT1 > bT0
        bestT = jnp.where(gtT, bT1, bT0)
        validT = (bestT > 0.5) & (
            lax.broadcasted_iota(jnp.int32, (NPAD, 1), 0) < N)
        idxT = jnp.where(gtT, pidxT_ref[0, :, 1:2], pidxT_ref[0, :, 0:1])
        scoT = jnp.where(gtT, pscoreT_ref[0, :, 1:2], pscoreT_ref[0, :, 0:1])
        idx_row = jnp.where(gt, pidx_ref[0, 1], pidx_ref[0, 0])[None, :]
        nsub = lax.broadcasted_iota(jnp.int32, (NPAD, NPAD), 0)
        jlane = lax.broadcasted_iota(jnp.int32, (NPAD, NPAD), 1)
        eq = idxT == idx_row
        later = jlane > nsub
        dup = jnp.any(eq & later & valid[None, :], axis=1, keepdims=True)
        winT = (validT & jnp.logical_not(dup)).astype(jnp.float32)
        dot = jnp.sum(winT * bestT * scoT)
        conf_v = (sum_bce0 - dot) / P

        acc_ref[0] += box_v
        acc_ref[1] += scale_v
        acc_ref[2] += ctx_v
        acc_ref[3] += conf_v

    @pl.when(cnt == 0.0)
    def _else_branch():
        box_e = jnp.sum(jnp.abs(boxesf_ref[0, 0, :])) / (P * 4.0) * 0.1
        sca = scalesf_ref[0, 0, :]
        ent = -(sca * jnp.log(sca + 1e-6))
        scale_e = jnp.sum(ent) / (P * S) * 0.1
        cx = ctx_ref[0, 0, :]
        cbce0 = jnp.maximum(cx, 0.0) + jnp.log1p(jnp.exp(-jnp.abs(cx)))
        ctx_e = jnp.sum(cbce0) / P * 0.1
        acc_ref[0] += box_e
        acc_ref[1] += scale_e
        acc_ref[2] += ctx_e
        acc_ref[3] += sum_bce0 / P

    @pl.when(i == B - 1)
    def _final():
        wb = 2.0 * acc_ref[0] / B
        ws = 1.0 * acc_ref[1] / B
        wc = 1.5 * acc_ref[2] / B
        wf = 1.0 * acc_ref[3] / B
        total = wb + ws + wc + wf
        bad = jnp.isnan(total) | jnp.isinf(total)
        total = jnp.where(bad, jnp.float32(0.1), total)
        r = lax.broadcasted_iota(jnp.int32, (8, 128), 0)
        col0 = lax.broadcasted_iota(jnp.int32, (8, 128), 1) == 0
        out = jnp.zeros((8, 128), jnp.float32)
        for row, val in enumerate((total, wb, ws, wc, wf)):
            out = jnp.where((r == row) & col0, val, out)
        res_ref[...] = out


def kernel(scores, boxes, scales, context_scores, target_boxes,
           target_scales, target_context, target_confidence):
    del target_confidence
    padN = NPAD - N
    f32 = jnp.float32

    tpadbox = jnp.broadcast_to(jnp.asarray([2.0, 2.0, 3.0, 3.0], f32),
                               (B, padN, 4))
    tb = jnp.concatenate([target_boxes, tpadbox], axis=1)
    tsc_p = jnp.pad(target_scales, ((0, 0), (0, padN))).astype(f32)
    tctx_p = jnp.pad(target_context, ((0, 0), (0, padN)))

    txf = [tb[:, :, i].reshape(B * NPAD) for i in range(4)]
    pxf = [boxes[:, :, i].reshape(B * P) for i in range(4)]
    scf = scores.reshape(B * P)
    ctxf = context_scores.reshape(B * P)
    scalesf = scales.reshape(B * P * 8)

    pbest, pidx, pbox, pscale, pctx, pscore = _sc_kernel(
        pxf[0], pxf[1], pxf[2], pxf[3], txf[0], txf[1], txf[2], txf[3],
        scf, ctxf, scalesf)

    pbest2 = pbest.reshape(B, 2, NPAD)
    pidx2 = pidx.reshape(B, 2, NPAD)
    pbox2 = pbox.reshape(B, 2, 4, NPAD)
    pscale2 = pscale.reshape(B, 2, 8, NPAD)
    pctx2 = pctx.reshape(B, 2, NPAD)
    pscore2 = pscore.reshape(B, 2, NPAD)
    pbestT = pbest2.transpose(0, 2, 1)
    pidxT = pidx2.transpose(0, 2, 1)
    pscoreT = pscore2.transpose(0, 2, 1)

    in_specs = [
        pl.BlockSpec((1, 1, P), lambda i: (i, 0, 0)),
        pl.BlockSpec((1, 1, P), lambda i: (i, 0, 0)),
        pl.BlockSpec((1, 1, P * S), lambda i: (i, 0, 0)),
        pl.BlockSpec((1, 1, P * 4), lambda i: (i, 0, 0)),
        pl.BlockSpec((1, 2, NPAD), lambda i: (i, 0, 0)),
        pl.BlockSpec((1, 2, NPAD), lambda i: (i, 0, 0)),
        pl.BlockSpec((1, NPAD, 2), lambda i: (i, 0, 0)),
        pl.BlockSpec((1, NPAD, 2), lambda i: (i, 0, 0)),
        pl.BlockSpec((1, NPAD, 2), lambda i: (i, 0, 0)),
        pl.BlockSpec((1, 2, 4, NPAD), lambda i: (i, 0, 0, 0)),
        pl.BlockSpec((1, 2, S, NPAD), lambda i: (i, 0, 0, 0)),
        pl.BlockSpec((1, 2, NPAD), lambda i: (i, 0, 0)),
        pl.BlockSpec((1, 4, NPAD), lambda i: (i, 0, 0)),
        pl.BlockSpec((1, 1, NPAD), lambda i: (i, 0, 0)),
        pl.BlockSpec((1, 1, NPAD), lambda i: (i, 0, 0)),
    ]

    res = pl.pallas_call(
        _tc_body,
        grid=(B,),
        in_specs=in_specs,
        out_specs=pl.BlockSpec((8, 128), lambda i: (0, 0)),
        out_shape=jax.ShapeDtypeStruct((8, 128), jnp.float32),
        scratch_shapes=[pltpu.SMEM((8,), jnp.float32)],
    )(
        scores.reshape(B, 1, P), context_scores.reshape(B, 1, P),
        scales.reshape(B, 1, P * S), boxes.reshape(B, 1, P * 4),
        pbest2, pidx2, pbestT, pidxT, pscoreT,
        pbox2, pscale2, pctx2,
        tb.transpose(0, 2, 1), tsc_p.reshape(B, 1, NPAD),
        tctx_p.reshape(B, 1, NPAD),
    )

    total = res[0, 0]
    wb = res[1, 0]
    ws = res[2, 0]
    wc = res[3, 0]
    wf = res[4, 0]
    return (total, wb, ws, wc, wf)


# speedup vs baseline: 1.2757x; 1.2757x over previous
import functools

import jax
import jax.numpy as jnp
from jax import lax
from jax.experimental import pallas as pl
from jax.experimental.pallas import tpu as pltpu
from jax.experimental.pallas import tpu_sc as plsc

B, P, N, S = 16, 5000, 100, 8
L = 16
PH = 2512
H1OFF = P - PH
NPAD = 112
NCHUNK = PH // L
TCH = NPAD // L
TGB = 4
NW = 32


def _sc_body(px0h, py0h, px1h, py1h, tx0h, ty0h, tx1h, ty1h,
             scoresh, ctxh, s0h, s1h, s2h, s3h, s4h, s5h, s6h, s7h,
             pbest, pidx, pbox, pscale, pctx, pscore,
             px0, py0, px1, py1, a1,
             tx0, ty0, tx1, ty1, a2,
             bval, bidx, gidx, buf14, sem):
    c = lax.axis_index("c")
    s = lax.axis_index("s")
    b = c * 8 + s // 2
    h = s % 2
    w = c * 16 + s
    poff = b * P + h * H1OFF
    toff = b * NPAD

    pltpu.sync_copy(px0h.at[pl.ds(poff, PH)], px0)
    pltpu.sync_copy(py0h.at[pl.ds(poff, PH)], py0)
    pltpu.sync_copy(px1h.at[pl.ds(poff, PH)], px1)
    pltpu.sync_copy(py1h.at[pl.ds(poff, PH)], py1)
    pltpu.sync_copy(tx0h.at[pl.ds(toff, NPAD)], tx0)
    pltpu.sync_copy(ty0h.at[pl.ds(toff, NPAD)], ty0)
    pltpu.sync_copy(tx1h.at[pl.ds(toff, NPAD)], tx1)
    pltpu.sync_copy(ty1h.at[pl.ds(toff, NPAD)], ty1)

    lane = lax.iota(jnp.int32, L)

    def _area1(i, _):
        sl = pl.ds(i * L, L)
        a1[sl] = (px1[sl] - px0[sl]) * (py1[sl] - py0[sl])
        return 0
    lax.fori_loop(0, NCHUNK, _area1, 0)

    def _area2(i, _):
        sl = pl.ds(i * L, L)
        a2[sl] = (tx1[sl] - tx0[sl]) * (ty1[sl] - ty0[sl])
        return 0
    lax.fori_loop(0, TCH, _area2, 0)

    minf = jnp.full((L,), -jnp.inf, jnp.float32)
    zi = jnp.zeros((L,), jnp.int32)

    def _per_tchunk(tc, _):
        tsl = pl.ds(tc * L, L)
        v0 = tx0[tsl]
        v1 = ty0[tsl]
        v2 = tx1[tsl]
        v3 = ty1[tsl]
        va = a2[tsl]

        def _per_group(g, carry):
            resv, residx = carry
            t0s, t1s, t2s, t3s, a2s = [], [], [], [], []
            for jj in range(TGB):
                sel = lane == (g * TGB + jj)
                t0s.append(jnp.sum(jnp.where(sel, v0, 0.0)))
                t1s.append(jnp.sum(jnp.where(sel, v1, 0.0)))
                t2s.append(jnp.sum(jnp.where(sel, v2, 0.0)))
                t3s.append(jnp.sum(jnp.where(sel, v3, 0.0)))
                a2s.append(jnp.sum(jnp.where(sel, va, 0.0)))

            def _chunk(k, carry2):
                ms = list(carry2[:TGB])
                bis = list(carry2[TGB:])
                sl = pl.ds(k * L, L)
                x0 = px0[sl]
                y0 = py0[sl]
                x1 = px1[sl]
                y1 = py1[sl]
                ar = a1[sl]
                for jj in range(TGB):
                    wx = jnp.maximum(
                        jnp.minimum(x1, t2s[jj]) - jnp.maximum(x0, t0s[jj]),
                        0.0)
                    wy = jnp.maximum(
                        jnp.minimum(y1, t3s[jj]) - jnp.maximum(y0, t1s[jj]),
                        0.0)
                    inter = wx * wy
                    iou = inter / ((ar + a2s[jj]) - inter)
                    gt = iou > ms[jj]
                    ms[jj] = jnp.where(gt, iou, ms[jj])
                    bis[jj] = jnp.where(gt, k, bis[jj])
                return tuple(ms) + tuple(bis)

            init = (minf,) * TGB + (zi,) * TGB
            out = lax.fori_loop(0, NCHUNK, _chunk, init)
            for jj in range(TGB):
                m = out[jj]
                bi = out[TGB + jj]
                vmax = jnp.max(m)
                pc = jnp.where(m == vmax, bi * L + lane, jnp.int32(2 ** 30))
                pmin = jnp.min(pc)
                pmin = jnp.where(pmin >= 2 ** 30, jnp.int32(0), pmin)
                sel = lane == (g * TGB + jj)
                resv = jnp.where(sel, vmax, resv)
                residx = jnp.where(sel, pmin + h * H1OFF, residx)
            return resv, residx

        z = jnp.zeros((L,), jnp.float32)
        resv, residx = lax.fori_loop(0, L // TGB, _per_group, (z, zi))
        bval[tsl] = resv
        bidx[tsl] = residx
        gidx[tsl] = residx + b * P
        return 0

    lax.fori_loop(0, TCH, _per_tchunk, 0)

    pltpu.sync_copy(bval, pbest.at[w])
    pltpu.sync_copy(bidx, pidx.at[w])

    copies = []

    def _fire(table, j):
        copies.append(
            pltpu.make_async_copy(table.at[gidx], buf14.at[j], sem))
        copies[-1].start()

    _fire(scoresh, 0)
    _fire(ctxh, 1)
    for cc, tbl in enumerate((px0h, py0h, px1h, py1h)):
        _fire(tbl, 2 + cc)
    for jj, tbl in enumerate((s0h, s1h, s2h, s3h, s4h, s5h, s6h, s7h)):
        _fire(tbl, 6 + jj)
    for cp in copies:
        cp.wait()

    pltpu.sync_copy(buf14.at[0], pscore.at[w])
    pltpu.sync_copy(buf14.at[1], pctx.at[w])
    for cc in range(4):
        pltpu.sync_copy(buf14.at[2 + cc], pbox.at[w, cc])
    for jj in range(8):
        pltpu.sync_copy(buf14.at[6 + jj], pscale.at[w, jj])


_sc_kernel = functools.partial(
    pl.kernel,
    out_type=[
        jax.ShapeDtypeStruct((NW, NPAD), jnp.float32),
        jax.ShapeDtypeStruct((NW, NPAD), jnp.int32),
        jax.ShapeDtypeStruct((NW, 4, NPAD), jnp.float32),
        jax.ShapeDtypeStruct((NW, 8, NPAD), jnp.float32),
        jax.ShapeDtypeStruct((NW, NPAD), jnp.float32),
        jax.ShapeDtypeStruct((NW, NPAD), jnp.float32),
    ],
    mesh=plsc.VectorSubcoreMesh(core_axis_name="c", subcore_axis_name="s",
                                num_cores=2, num_subcores=16),
    compiler_params=pltpu.CompilerParams(needs_layout_passes=False),
    scratch_types=[
        pltpu.VMEM((PH,), jnp.float32),
        pltpu.VMEM((PH,), jnp.float32),
        pltpu.VMEM((PH,), jnp.float32),
        pltpu.VMEM((PH,), jnp.float32),
        pltpu.VMEM((PH,), jnp.float32),
        pltpu.VMEM((NPAD,), jnp.float32),
        pltpu.VMEM((NPAD,), jnp.float32),
        pltpu.VMEM((NPAD,), jnp.float32),
        pltpu.VMEM((NPAD,), jnp.float32),
        pltpu.VMEM((NPAD,), jnp.float32),
        pltpu.VMEM((NPAD,), jnp.float32),
        pltpu.VMEM((NPAD,), jnp.int32),
        pltpu.VMEM((NPAD,), jnp.int32),
        pltpu.VMEM((14, NPAD), jnp.float32),
        pltpu.SemaphoreType.DMA,
    ],
)(_sc_body)


def _tc_body(scores_ref, ctx_ref, scalesf_ref, boxesf_ref,
             pbest_ref, pidx_ref, pbestT_ref, pidxT_ref, pscoreT_ref,
             pbox_ref, pscale_ref, pctx_ref,
             tbox_ref, tscf_ref, tctx_ref,
             res_ref, acc_ref):
    i = pl.program_id(0)

    @pl.when(i == 0)
    def _init():
        res_ref[...] = jnp.zeros((8, 128), jnp.float32)
        acc_ref[0] = 0.0
        acc_ref[1] = 0.0
        acc_ref[2] = 0.0
        acc_ref[3] = 0.0

    sc = scores_ref[0, 0, :]
    bce0 = jnp.maximum(sc, 0.0) + jnp.log1p(jnp.exp(-jnp.abs(sc)))
    sum_bce0 = jnp.sum(bce0)

    v0 = pbest_ref[0, 0, :]
    v1 = pbest_ref[0, 1, :]
    gt = v1 > v0
    best = jnp.where(gt, v1, v0)
    nmask = lax.broadcasted_iota(jnp.int32, (NPAD,), 0) < N
    valid = (best > 0.5) & nmask
    vf = valid.astype(jnp.float32)
    cnt = jnp.sum(vf)
    cnt_s = jnp.maximum(cnt, 1.0)

    @pl.when(cnt > 0.0)
    def _valid_branch():
        gtb = gt[None, :]
        sel_box = jnp.where(gtb, pbox_ref[0, 1], pbox_ref[0, 0])
        tbox = tbox_ref[0]
        d = jnp.abs(sel_box - tbox)
        sl1 = jnp.where(d < 0.1, 5.0 * d * d, d - 0.05)
        bl = jnp.sum(sl1, axis=0) * best
        box_v = jnp.sum(bl * vf) / (cnt_s * 4.0)

        osc = jnp.where(gtb, pscale_ref[0, 1], pscale_ref[0, 0])
        mx = jnp.max(osc, axis=0, keepdims=True)
        lse = mx[0] + jnp.log(jnp.sum(jnp.exp(osc - mx), axis=0))
        srange = lax.broadcasted_iota(jnp.int32, (S, NPAD), 0).astype(jnp.float32)
        oh = (srange == tscf_ref[0]).astype(jnp.float32)
        picked = jnp.sum(osc * oh, axis=0)
        scale_v = jnp.sum((lse - picked) * vf) / cnt_s

        octx = jnp.where(gt, pctx_ref[0, 1], pctx_ref[0, 0])
        tctx = tctx_ref[0, 0, :]
        cbce = (jnp.maximum(octx, 0.0) - octx * tctx
                + jnp.log1p(jnp.exp(-jnp.abs(octx))))
        ctx_v = jnp.sum(cbce * vf) / cnt_s

        bT0 = pbestT_ref[0, :, 0:1]
        bT1 = pbestT_ref[0, :, 1:2]
        gtT = bT1 > bT0
        bestT = jnp.where(gtT, bT1, bT0)
        validT = (bestT > 0.5) & (
            lax.broadcasted_iota(jnp.int32, (NPAD, 1), 0) < N)
        idxT = jnp.where(gtT, pidxT_ref[0, :, 1:2], pidxT_ref[0, :, 0:1])
        scoT = jnp.where(gtT, pscoreT_ref[0, :, 1:2], pscoreT_ref[0, :, 0:1])
        idx_row = jnp.where(gt, pidx_ref[0, 1], pidx_ref[0, 0])[None, :]
        nsub = lax.broadcasted_iota(jnp.int32, (NPAD, NPAD), 0)
        jlane = lax.broadcasted_iota(jnp.int32, (NPAD, NPAD), 1)
        eq = idxT == idx_row
        later = jlane > nsub
        dup = jnp.any(eq & later & valid[None, :], axis=1, keepdims=True)
        winT = (validT & jnp.logical_not(dup)).astype(jnp.float32)
        dot = jnp.sum(winT * bestT * scoT)
        conf_v = (sum_bce0 - dot) / P

        acc_ref[0] += box_v
        acc_ref[1] += scale_v
        acc_ref[2] += ctx_v
        acc_ref[3] += conf_v

    @pl.when(cnt == 0.0)
    def _else_branch():
        box_e = jnp.sum(jnp.abs(boxesf_ref[0, 0, :])) / (P * 4.0) * 0.1
        sca = scalesf_ref[0, 0, :]
        ent = -(sca * jnp.log(sca + 1e-6))
        scale_e = jnp.sum(ent) / (P * S) * 0.1
        cx = ctx_ref[0, 0, :]
        cbce0 = jnp.maximum(cx, 0.0) + jnp.log1p(jnp.exp(-jnp.abs(cx)))
        ctx_e = jnp.sum(cbce0) / P * 0.1
        acc_ref[0] += box_e
        acc_ref[1] += scale_e
        acc_ref[2] += ctx_e
        acc_ref[3] += sum_bce0 / P

    @pl.when(i == B - 1)
    def _final():
        wb = 2.0 * acc_ref[0] / B
        ws = 1.0 * acc_ref[1] / B
        wc = 1.5 * acc_ref[2] / B
        wf = 1.0 * acc_ref[3] / B
        total = wb + ws + wc + wf
        bad = jnp.isnan(total) | jnp.isinf(total)
        total = jnp.where(bad, jnp.float32(0.1), total)
        r = lax.broadcasted_iota(jnp.int32, (8, 128), 0)
        col0 = lax.broadcasted_iota(jnp.int32, (8, 128), 1) == 0
        out = jnp.zeros((8, 128), jnp.float32)
        for row, val in enumerate((total, wb, ws, wc, wf)):
            out = jnp.where((r == row) & col0, val, out)
        res_ref[...] = out


def kernel(scores, boxes, scales, context_scores, target_boxes,
           target_scales, target_context, target_confidence):
    del target_confidence
    padN = NPAD - N
    f32 = jnp.float32

    tpadbox = jnp.broadcast_to(jnp.asarray([2.0, 2.0, 3.0, 3.0], f32),
                               (B, padN, 4))
    tb = jnp.concatenate([target_boxes, tpadbox], axis=1)
    tsc_p = jnp.pad(target_scales, ((0, 0), (0, padN))).astype(f32)
    tctx_p = jnp.pad(target_context, ((0, 0), (0, padN)))

    txf = [tb[:, :, i].reshape(B * NPAD) for i in range(4)]
    pxf = [boxes[:, :, i].reshape(B * P) for i in range(4)]
    scf = scores.reshape(B * P)
    ctxf = context_scores.reshape(B * P)
    splanes = [scales[:, :, j].reshape(B * P) for j in range(8)]

    pbest, pidx, pbox, pscale, pctx, pscore = _sc_kernel(
        pxf[0], pxf[1], pxf[2], pxf[3], txf[0], txf[1], txf[2], txf[3],
        scf, ctxf, *splanes)

    pbest2 = pbest.reshape(B, 2, NPAD)
    pidx2 = pidx.reshape(B, 2, NPAD)
    pbox2 = pbox.reshape(B, 2, 4, NPAD)
    pscale2 = pscale.reshape(B, 2, 8, NPAD)
    pctx2 = pctx.reshape(B, 2, NPAD)
    pscore2 = pscore.reshape(B, 2, NPAD)
    pbestT = pbest2.transpose(0, 2, 1)
    pidxT = pidx2.transpose(0, 2, 1)
    pscoreT = pscore2.transpose(0, 2, 1)

    in_specs = [
        pl.BlockSpec((1, 1, P), lambda i: (i, 0, 0)),
        pl.BlockSpec((1, 1, P), lambda i: (i, 0, 0)),
        pl.BlockSpec((1, 1, P * S), lambda i: (i, 0, 0)),
        pl.BlockSpec((1, 1, P * 4), lambda i: (i, 0, 0)),
        pl.BlockSpec((1, 2, NPAD), lambda i: (i, 0, 0)),
        pl.BlockSpec((1, 2, NPAD), lambda i: (i, 0, 0)),
        pl.BlockSpec((1, NPAD, 2), lambda i: (i, 0, 0)),
        pl.BlockSpec((1, NPAD, 2), lambda i: (i, 0, 0)),
        pl.BlockSpec((1, NPAD, 2), lambda i: (i, 0, 0)),
        pl.BlockSpec((1, 2, 4, NPAD), lambda i: (i, 0, 0, 0)),
        pl.BlockSpec((1, 2, S, NPAD), lambda i: (i, 0, 0, 0)),
        pl.BlockSpec((1, 2, NPAD), lambda i: (i, 0, 0)),
        pl.BlockSpec((1, 4, NPAD), lambda i: (i, 0, 0)),
        pl.BlockSpec((1, 1, NPAD), lambda i: (i, 0, 0)),
        pl.BlockSpec((1, 1, NPAD), lambda i: (i, 0, 0)),
    ]

    res = pl.pallas_call(
        _tc_body,
        grid=(B,),
        in_specs=in_specs,
        out_specs=pl.BlockSpec((8, 128), lambda i: (0, 0)),
        out_shape=jax.ShapeDtypeStruct((8, 128), jnp.float32),
        scratch_shapes=[pltpu.SMEM((8,), jnp.float32)],
    )(
        scores.reshape(B, 1, P), context_scores.reshape(B, 1, P),
        scales.reshape(B, 1, P * S), boxes.reshape(B, 1, P * 4),
        pbest2, pidx2, pbestT, pidxT, pscoreT,
        pbox2, pscale2, pctx2,
        tb.transpose(0, 2, 1), tsc_p.reshape(B, 1, NPAD),
        tctx_p.reshape(B, 1, NPAD),
    )

    total = res[0, 0]
    wb = res[1, 0]
    ws = res[2, 0]
    wc = res[3, 0]
    wf = res[4, 0]
    return (total, wb, ws, wc, wf)
